# baseline (speedup 1.0000x reference)
import dataclasses
import functools

import jax
import jax.numpy as jnp
from jax import lax
from jax.experimental import pallas as pl
from jax.experimental.pallas import tpu as pltpu
from jax.experimental.pallas import tpu_sc as plsc

ROWS = 50000
FEAT = 256
HALF = FEAT // 2
NSEG = 1024
CHUNK = 80
NCHUNKS = ROWS // CHUNK
NC, NS, LANES = 2, 16, 16
NW = NC * NS
TROWS = NSEG // NS

TCCHUNK = 2000
RTC = 26000
NTCCHUNK = RTC // TCCHUNK
CTC = RTC // CHUNK
CSLICE = next(s for s in range(RTC // NW // 16 * 16, RTC, 16)
              if 0 < RTC - (NW - 1) * s <= s and (RTC - (NW - 1) * s) % 16 == 0)
CSLICE_LAST = RTC - (NW - 1) * CSLICE

_mesh = plsc.VectorSubcoreMesh(core_axis_name="c", subcore_axis_name="s")

_sc_params = pltpu.CompilerParams()
if "needs_layout_passes" in pltpu.CompilerParams.__dataclass_fields__:
    _sc_params = dataclasses.replace(_sc_params, needs_layout_passes=False)


@functools.partial(
    pl.kernel,
    mesh=_mesh,
    out_type=[
        jax.ShapeDtypeStruct((NC, NSEG, FEAT), jnp.float32),
        jax.ShapeDtypeStruct((NW, NSEG), jnp.float32),
    ],
    compiler_params=_sc_params,
    scratch_types=[
        pltpu.VMEM((2, CHUNK, HALF), jnp.float32),
        pltpu.VMEM((2, CHUNK, HALF), jnp.float32),
        pltpu.VMEM((CHUNK,), jnp.int32),
        pltpu.VMEM((CHUNK,), jnp.int32),
        pltpu.VMEM((NSEG,), jnp.float32),
        pltpu.VMEM((CSLICE,), jnp.int32),
        pltpu.VMEM((TROWS, HALF), jnp.float32),
        pltpu.VMEM_SHARED((NSEG, HALF), jnp.float32),
        pltpu.VMEM_SHARED((NSEG, HALF), jnp.float32),
        pltpu.SemaphoreType.DMA,
        pltpu.SemaphoreType.DMA,
    ],
)
def _sc_partial(x_hbm, b_hbm, sums_hbm, cnts_hbm, xl, xr, bbuf0, bbuf1, hist,
                cbuf, zstage, accl, accr, sem0, sem1):
    c = lax.axis_index("c")
    s = lax.axis_index("s")
    w = c * NS + s
    bbufs = (bbuf0, bbuf1)
    sems = (sem0, sem1)

    one = jnp.ones((LANES,), jnp.float32)
    zero = jnp.zeros((LANES,), jnp.float32)

    @pl.loop(0, NSEG // LANES)
    def _(i):
        hist[pl.ds(i * LANES, LANES)] = zero

    @pl.loop(0, TROWS)
    def _(i):
        @pl.loop(0, HALF // LANES)
        def _(j):
            zstage[i, pl.ds(j * LANES, LANES)] = zero

    row = s * TROWS
    pltpu.sync_copy(zstage, accl.at[pl.ds(row, TROWS)])
    pltpu.sync_copy(zstage, accr.at[pl.ds(row, TROWS)])
    plsc.subcore_barrier()

    nsc = NCHUNKS - CTC
    k0 = CTC + w * nsc // NW
    k1 = CTC + (w + 1) * nsc // NW
    cntc = k1 - k0

    def start(k, slot):
        r = k * CHUNK
        sem = sems[slot]
        cps = (
            pltpu.make_async_copy(
                x_hbm.at[pl.ds(r, CHUNK), pl.ds(0, HALF)], xl.at[slot], sem),
            pltpu.make_async_copy(
                x_hbm.at[pl.ds(r, CHUNK), pl.ds(HALF, HALF)], xr.at[slot], sem),
            pltpu.make_async_copy(b_hbm.at[pl.ds(r, CHUNK)], bbufs[slot], sem),
        )
        for cp in cps:
            cp.start()
        return cps

    def wait(cps):
        for cp in cps:
            cp.wait()

    def accumulate(slot):
        pltpu.sync_copy(xl.at[slot], accl.at[bbufs[slot]], add=True)
        pltpu.sync_copy(xr.at[slot], accr.at[bbufs[slot]], add=True)

        @pl.loop(0, CHUNK // LANES)
        def _(i):
            plsc.addupdate_scatter(
                hist, [bbufs[slot][pl.ds(i * LANES, LANES)]], one)

    cp0 = start(k0, 0)
    npairs = cntc // 2

    @pl.loop(0, npairs)
    def _(p):
        k = k0 + 2 * p
        nxt = start(k + 1, 1)
        wait(cp0)
        accumulate(0)

        @pl.when(k + 2 < k1)
        def _():
            start(k + 2, 0)

        wait(nxt)
        accumulate(1)

    @pl.when(k0 + 2 * npairs < k1)
    def _():
        wait(cp0)
        accumulate(0)

    cb = w * CSLICE

    @pl.when(w < NW - 1)
    def _():
        pltpu.sync_copy(b_hbm.at[pl.ds(cb, CSLICE)], cbuf)

    @pl.when(w == NW - 1)
    def _():
        pltpu.sync_copy(b_hbm.at[pl.ds(cb, CSLICE_LAST)],
                        cbuf.at[pl.ds(0, CSLICE_LAST)])

    nit = jnp.where(w == NW - 1, CSLICE_LAST // LANES, CSLICE // LANES)

    @pl.loop(0, nit)
    def _(i):
        plsc.addupdate_scatter(hist, [cbuf[pl.ds(i * LANES, LANES)]], one)

    pltpu.sync_copy(hist, cnts_hbm.at[w])
    plsc.subcore_barrier()
    pltpu.sync_copy(accl.at[pl.ds(row, TROWS)],
                    sums_hbm.at[c, pl.ds(row, TROWS), pl.ds(0, HALF)])
    pltpu.sync_copy(accr.at[pl.ds(row, TROWS)],
                    sums_hbm.at[c, pl.ds(row, TROWS), pl.ds(HALF, HALF)])


def _sums_body(b_ref, x_ref, o_ref):
    i = pl.program_id(0)
    bvec = b_ref[0, 0, :].astype(jnp.int16)
    gids = jax.lax.broadcasted_iota(jnp.int16, (NSEG, TCCHUNK), 0)
    onehot = (gids == bvec[None, :]).astype(jnp.bfloat16)
    psum = jax.lax.dot(onehot, x_ref[...].astype(jnp.bfloat16),
                       preferred_element_type=jnp.float32)

    @pl.when(i == 0)
    def _():
        o_ref[...] = psum

    @pl.when(i > 0)
    def _():
        o_ref[...] += psum


def _tc_sums(x, b3):
    return pl.pallas_call(
        _sums_body,
        grid=(NTCCHUNK,),
        in_specs=[
            pl.BlockSpec((1, 1, TCCHUNK), lambda i: (i, 0, 0)),
            pl.BlockSpec((TCCHUNK, FEAT), lambda i: (i, 0)),
        ],
        out_specs=pl.BlockSpec((NSEG, FEAT), lambda i: (0, 0)),
        out_shape=jax.ShapeDtypeStruct((NSEG, FEAT), jnp.float32),
    )(b3, x)


def _combine_body(ts_ref, sp_ref, cp_ref, o_ref):
    ssum = ts_ref[...] + sp_ref[0] + sp_ref[1]
    csum = jnp.sum(cp_ref[...], axis=0).reshape(NSEG, 1)
    o_ref[...] = ssum / jnp.maximum(csum, 1.0)


def kernel(x, batch):
    b = batch.astype(jnp.int32)
    b3 = b.reshape(ROWS // TCCHUNK, 1, TCCHUNK)
    tc_sums = _tc_sums(x, b3)
    sums, cnts = _sc_partial(x, b)
    return pl.pallas_call(
        _combine_body,
        out_shape=jax.ShapeDtypeStruct((NSEG, FEAT), jnp.float32),
    )(tc_sums, sums, cnts)


# speedup vs baseline: 1.0128x; 1.0128x over previous
import dataclasses
import functools

import jax
import jax.numpy as jnp
from jax import lax
from jax.experimental import pallas as pl
from jax.experimental.pallas import tpu as pltpu
from jax.experimental.pallas import tpu_sc as plsc

ROWS = 50000
FEAT = 256
HALF = FEAT // 2
NSEG = 1024
CHUNK = 80
NCHUNKS = ROWS // CHUNK
NC, NS, LANES = 2, 16, 16
NW = NC * NS
TROWS = NSEG // NS

TCCHUNK = 2000
RTC = 26000
NTCCHUNK = RTC // TCCHUNK
CTC = RTC // CHUNK
CSLICE = next(s for s in range(RTC // NW // 16 * 16, RTC, 16)
              if 0 < RTC - (NW - 1) * s <= s and (RTC - (NW - 1) * s) % 16 == 0)
CSLICE_LAST = RTC - (NW - 1) * CSLICE

_mesh = plsc.VectorSubcoreMesh(core_axis_name="c", subcore_axis_name="s")

_sc_params = pltpu.CompilerParams()
if "needs_layout_passes" in pltpu.CompilerParams.__dataclass_fields__:
    _sc_params = dataclasses.replace(_sc_params, needs_layout_passes=False)


@functools.partial(
    pl.kernel,
    mesh=_mesh,
    out_type=[
        jax.ShapeDtypeStruct((NC, NSEG, FEAT), jnp.float32),
        jax.ShapeDtypeStruct((NW, NSEG), jnp.float32),
    ],
    compiler_params=_sc_params,
    scratch_types=[
        pltpu.VMEM((2, CHUNK, HALF), jnp.float32),
        pltpu.VMEM((2, CHUNK, HALF), jnp.float32),
        pltpu.VMEM((CHUNK,), jnp.int32),
        pltpu.VMEM((CHUNK,), jnp.int32),
        pltpu.VMEM((NSEG,), jnp.float32),
        pltpu.VMEM((CSLICE,), jnp.int32),
        pltpu.VMEM((TROWS, HALF), jnp.float32),
        pltpu.VMEM_SHARED((NSEG, HALF), jnp.float32),
        pltpu.VMEM_SHARED((NSEG, HALF), jnp.float32),
        pltpu.SemaphoreType.DMA,
        pltpu.SemaphoreType.DMA,
    ],
)
def _sc_partial(x_hbm, b_hbm, sums_hbm, cnts_hbm, xl, xr, bbuf0, bbuf1, hist,
                cbuf, zstage, accl, accr, sem0, sem1):
    c = lax.axis_index("c")
    s = lax.axis_index("s")
    w = c * NS + s
    bbufs = (bbuf0, bbuf1)
    sems = (sem0, sem1)

    one = jnp.ones((LANES,), jnp.float32)
    zero = jnp.zeros((LANES,), jnp.float32)

    @pl.loop(0, NSEG // LANES)
    def _(i):
        hist[pl.ds(i * LANES, LANES)] = zero

    @pl.loop(0, TROWS)
    def _(i):
        @pl.loop(0, HALF // LANES)
        def _(j):
            zstage[i, pl.ds(j * LANES, LANES)] = zero

    row = s * TROWS
    pltpu.sync_copy(zstage, accl.at[pl.ds(row, TROWS)])
    pltpu.sync_copy(zstage, accr.at[pl.ds(row, TROWS)])
    plsc.subcore_barrier()

    nsc = NCHUNKS - CTC
    k0 = CTC + w * nsc // NW
    k1 = CTC + (w + 1) * nsc // NW
    cntc = k1 - k0

    def start(k, slot):
        r = k * CHUNK
        sem = sems[slot]
        cps = (
            pltpu.make_async_copy(
                x_hbm.at[pl.ds(r, CHUNK), pl.ds(0, HALF)], xl.at[slot], sem),
            pltpu.make_async_copy(
                x_hbm.at[pl.ds(r, CHUNK), pl.ds(HALF, HALF)], xr.at[slot], sem),
            pltpu.make_async_copy(b_hbm.at[pl.ds(r, CHUNK)], bbufs[slot], sem),
        )
        for cp in cps:
            cp.start()
        return cps

    def wait(cps):
        for cp in cps:
            cp.wait()

    def accumulate(slot):
        pltpu.sync_copy(xl.at[slot], accl.at[bbufs[slot]], add=True)
        pltpu.sync_copy(xr.at[slot], accr.at[bbufs[slot]], add=True)

        @pl.loop(0, CHUNK // LANES)
        def _(i):
            plsc.addupdate_scatter(
                hist, [bbufs[slot][pl.ds(i * LANES, LANES)]], one)

    cp0 = start(k0, 0)
    npairs = cntc // 2

    @pl.loop(0, npairs)
    def _(p):
        k = k0 + 2 * p
        nxt = start(k + 1, 1)
        wait(cp0)
        accumulate(0)

        @pl.when(k + 2 < k1)
        def _():
            start(k + 2, 0)

        wait(nxt)
        accumulate(1)

    @pl.when(k0 + 2 * npairs < k1)
    def _():
        wait(cp0)
        accumulate(0)

    cb = w * CSLICE

    @pl.when(w < NW - 1)
    def _():
        pltpu.sync_copy(b_hbm.at[pl.ds(cb, CSLICE)], cbuf)

    @pl.when(w == NW - 1)
    def _():
        pltpu.sync_copy(b_hbm.at[pl.ds(cb, CSLICE_LAST)],
                        cbuf.at[pl.ds(0, CSLICE_LAST)])

    nit = jnp.where(w == NW - 1, CSLICE_LAST // LANES, CSLICE // LANES)

    @pl.loop(0, nit)
    def _(i):
        plsc.addupdate_scatter(hist, [cbuf[pl.ds(i * LANES, LANES)]], one)

    pltpu.sync_copy(hist, cnts_hbm.at[w])
    plsc.subcore_barrier()
    pltpu.sync_copy(accl.at[pl.ds(row, TROWS)],
                    sums_hbm.at[c, pl.ds(row, TROWS), pl.ds(0, HALF)])
    pltpu.sync_copy(accr.at[pl.ds(row, TROWS)],
                    sums_hbm.at[c, pl.ds(row, TROWS), pl.ds(HALF, HALF)])


def _sums_body(b_ref, x_ref, o_ref):
    i = pl.program_id(0)
    bvec = b_ref[0, 0, :]
    gids = jax.lax.broadcasted_iota(jnp.int32, (NSEG, TCCHUNK), 0)
    onehot = (gids == bvec[None, :]).astype(jnp.bfloat16)
    psum = jax.lax.dot(onehot, x_ref[...].astype(jnp.bfloat16),
                       preferred_element_type=jnp.float32)

    @pl.when(i == 0)
    def _():
        o_ref[...] = psum

    @pl.when(i > 0)
    def _():
        o_ref[...] += psum


def _tc_sums(x, b3):
    return pl.pallas_call(
        _sums_body,
        grid=(NTCCHUNK,),
        in_specs=[
            pl.BlockSpec((1, 1, TCCHUNK), lambda i: (i, 0, 0)),
            pl.BlockSpec((TCCHUNK, FEAT), lambda i: (i, 0)),
        ],
        out_specs=pl.BlockSpec((NSEG, FEAT), lambda i: (0, 0)),
        out_shape=jax.ShapeDtypeStruct((NSEG, FEAT), jnp.float32),
    )(b3, x)


def _combine_body(ts_ref, sp_ref, cp_ref, o_ref):
    ssum = ts_ref[...] + sp_ref[0] + sp_ref[1]
    csum = jnp.sum(cp_ref[...], axis=0).reshape(NSEG, 1)
    o_ref[...] = ssum / jnp.maximum(csum, 1.0)


def kernel(x, batch):
    b = batch.astype(jnp.int32)
    b3 = b.reshape(ROWS // TCCHUNK, 1, TCCHUNK)
    tc_sums = _tc_sums(x, b3)
    sums, cnts = _sc_partial(x, b)
    return pl.pallas_call(
        _combine_body,
        out_shape=jax.ShapeDtypeStruct((NSEG, FEAT), jnp.float32),
    )(tc_sums, sums, cnts)


# speedup vs baseline: 1.0177x; 1.0048x over previous
import dataclasses
import functools

import jax
import jax.numpy as jnp
from jax import lax
from jax.experimental import pallas as pl
from jax.experimental.pallas import tpu as pltpu
from jax.experimental.pallas import tpu_sc as plsc

ROWS = 50000
FEAT = 256
HALF = FEAT // 2
NSEG = 1024
CHUNK = 96
NC, NS, LANES = 2, 16, 16
NW = NC * NS
TROWS = NSEG // NS

TCCHUNK = 2000
RTC = 26000
NTCCHUNK = RTC // TCCHUNK
NSC = (ROWS - RTC) // CHUNK
assert NSC * CHUNK == ROWS - RTC
CSLICE = next(s for s in range(RTC // NW // 16 * 16, RTC, 16)
              if 0 < RTC - (NW - 1) * s <= s and (RTC - (NW - 1) * s) % 16 == 0)
CSLICE_LAST = RTC - (NW - 1) * CSLICE

_mesh = plsc.VectorSubcoreMesh(core_axis_name="c", subcore_axis_name="s")

_sc_params = pltpu.CompilerParams()
if "needs_layout_passes" in pltpu.CompilerParams.__dataclass_fields__:
    _sc_params = dataclasses.replace(_sc_params, needs_layout_passes=False)


@functools.partial(
    pl.kernel,
    mesh=_mesh,
    out_type=[
        jax.ShapeDtypeStruct((NC, NSEG, FEAT), jnp.float32),
        jax.ShapeDtypeStruct((NW, NSEG), jnp.float32),
    ],
    compiler_params=_sc_params,
    scratch_types=[
        pltpu.VMEM((2, CHUNK, HALF), jnp.float32),
        pltpu.VMEM((2, CHUNK, HALF), jnp.float32),
        pltpu.VMEM((CHUNK,), jnp.int32),
        pltpu.VMEM((CHUNK,), jnp.int32),
        pltpu.VMEM((NSEG,), jnp.float32),
        pltpu.VMEM((CSLICE,), jnp.int32),
        pltpu.VMEM((TROWS, HALF), jnp.float32),
        pltpu.VMEM_SHARED((NSEG, HALF), jnp.float32),
        pltpu.VMEM_SHARED((NSEG, HALF), jnp.float32),
        pltpu.SemaphoreType.DMA,
        pltpu.SemaphoreType.DMA,
    ],
)
def _sc_partial(x_hbm, b_hbm, sums_hbm, cnts_hbm, xl, xr, bbuf0, bbuf1, hist,
                cbuf, zstage, accl, accr, sem0, sem1):
    c = lax.axis_index("c")
    s = lax.axis_index("s")
    w = c * NS + s
    bbufs = (bbuf0, bbuf1)
    sems = (sem0, sem1)

    one = jnp.ones((LANES,), jnp.float32)
    zero = jnp.zeros((LANES,), jnp.float32)

    @pl.loop(0, NSEG // LANES)
    def _(i):
        hist[pl.ds(i * LANES, LANES)] = zero

    @pl.loop(0, TROWS)
    def _(i):
        @pl.loop(0, HALF // LANES)
        def _(j):
            zstage[i, pl.ds(j * LANES, LANES)] = zero

    row = s * TROWS
    pltpu.sync_copy(zstage, accl.at[pl.ds(row, TROWS)])
    pltpu.sync_copy(zstage, accr.at[pl.ds(row, TROWS)])
    plsc.subcore_barrier()

    k0 = w * NSC // NW
    k1 = (w + 1) * NSC // NW
    cntc = k1 - k0

    def start(k, slot):
        r = RTC + k * CHUNK
        sem = sems[slot]
        cps = (
            pltpu.make_async_copy(
                x_hbm.at[pl.ds(r, CHUNK), pl.ds(0, HALF)], xl.at[slot], sem),
            pltpu.make_async_copy(
                x_hbm.at[pl.ds(r, CHUNK), pl.ds(HALF, HALF)], xr.at[slot], sem),
            pltpu.make_async_copy(b_hbm.at[pl.ds(r, CHUNK)], bbufs[slot], sem),
        )
        for cp in cps:
            cp.start()
        return cps

    def wait(cps):
        for cp in cps:
            cp.wait()

    def accumulate(slot):
        pltpu.sync_copy(xl.at[slot], accl.at[bbufs[slot]], add=True)
        pltpu.sync_copy(xr.at[slot], accr.at[bbufs[slot]], add=True)

        @pl.loop(0, CHUNK // LANES)
        def _(i):
            plsc.addupdate_scatter(
                hist, [bbufs[slot][pl.ds(i * LANES, LANES)]], one)

    cp0 = start(k0, 0)
    npairs = cntc // 2

    @pl.loop(0, npairs)
    def _(p):
        k = k0 + 2 * p
        nxt = start(k + 1, 1)
        wait(cp0)
        accumulate(0)

        @pl.when(k + 2 < k1)
        def _():
            start(k + 2, 0)

        wait(nxt)
        accumulate(1)

    @pl.when(k0 + 2 * npairs < k1)
    def _():
        wait(cp0)
        accumulate(0)

    cb = w * CSLICE

    @pl.when(w < NW - 1)
    def _():
        pltpu.sync_copy(b_hbm.at[pl.ds(cb, CSLICE)], cbuf)

    @pl.when(w == NW - 1)
    def _():
        pltpu.sync_copy(b_hbm.at[pl.ds(cb, CSLICE_LAST)],
                        cbuf.at[pl.ds(0, CSLICE_LAST)])

    nit = jnp.where(w == NW - 1, CSLICE_LAST // LANES, CSLICE // LANES)

    @pl.loop(0, nit)
    def _(i):
        plsc.addupdate_scatter(hist, [cbuf[pl.ds(i * LANES, LANES)]], one)

    pltpu.sync_copy(hist, cnts_hbm.at[w])
    plsc.subcore_barrier()
    pltpu.sync_copy(accl.at[pl.ds(row, TROWS)],
                    sums_hbm.at[c, pl.ds(row, TROWS), pl.ds(0, HALF)])
    pltpu.sync_copy(accr.at[pl.ds(row, TROWS)],
                    sums_hbm.at[c, pl.ds(row, TROWS), pl.ds(HALF, HALF)])


def _sums_body(b_ref, x_ref, o_ref):
    i = pl.program_id(0)
    bvec = b_ref[0, 0, :]
    gids = jax.lax.broadcasted_iota(jnp.int32, (NSEG, TCCHUNK), 0)
    onehot = (gids == bvec[None, :]).astype(jnp.bfloat16)
    psum = jax.lax.dot(onehot, x_ref[...].astype(jnp.bfloat16),
                       preferred_element_type=jnp.float32)

    @pl.when(i == 0)
    def _():
        o_ref[...] = psum

    @pl.when(i > 0)
    def _():
        o_ref[...] += psum


def _tc_sums(x, b3):
    return pl.pallas_call(
        _sums_body,
        grid=(NTCCHUNK,),
        in_specs=[
            pl.BlockSpec((1, 1, TCCHUNK), lambda i: (i, 0, 0)),
            pl.BlockSpec((TCCHUNK, FEAT), lambda i: (i, 0)),
        ],
        out_specs=pl.BlockSpec((NSEG, FEAT), lambda i: (0, 0)),
        out_shape=jax.ShapeDtypeStruct((NSEG, FEAT), jnp.float32),
    )(b3, x)


def _combine_body(ts_ref, sp_ref, cp_ref, o_ref):
    ssum = ts_ref[...] + sp_ref[0] + sp_ref[1]
    csum = jnp.sum(cp_ref[...], axis=0).reshape(NSEG, 1)
    o_ref[...] = ssum / jnp.maximum(csum, 1.0)


def kernel(x, batch):
    b = batch.astype(jnp.int32)
    b3 = b.reshape(ROWS // TCCHUNK, 1, TCCHUNK)
    tc_sums = _tc_sums(x, b3)
    sums, cnts = _sc_partial(x, b)
    return pl.pallas_call(
        _combine_body,
        out_shape=jax.ShapeDtypeStruct((NSEG, FEAT), jnp.float32),
    )(tc_sums, sums, cnts)


# speedup vs baseline: 1.0373x; 1.0193x over previous
import dataclasses
import functools

import jax
import jax.numpy as jnp
from jax import lax
from jax.experimental import pallas as pl
from jax.experimental.pallas import tpu as pltpu
from jax.experimental.pallas import tpu_sc as plsc

ROWS = 50000
FEAT = 256
HALF = FEAT // 2
NSEG = 1024
CHUNK = 96
NC, NS, LANES = 2, 16, 16
NW = NC * NS
TROWS = NSEG // NS

TCCHUNK = 2000
RTC = 26000
NTCCHUNK = RTC // TCCHUNK
NSC = (ROWS - RTC) // CHUNK
assert NSC * CHUNK == ROWS - RTC
CSLICE = next(s for s in range(RTC // NW // 16 * 16, RTC, 16)
              if 0 < RTC - (NW - 1) * s <= s and (RTC - (NW - 1) * s) % 16 == 0)
CSLICE_LAST = RTC - (NW - 1) * CSLICE

_mesh = plsc.VectorSubcoreMesh(core_axis_name="c", subcore_axis_name="s")

_sc_params = pltpu.CompilerParams()
if "needs_layout_passes" in pltpu.CompilerParams.__dataclass_fields__:
    _sc_params = dataclasses.replace(_sc_params, needs_layout_passes=False)


@functools.partial(
    pl.kernel,
    mesh=_mesh,
    out_type=[
        jax.ShapeDtypeStruct((NC, NSEG, FEAT), jnp.float32),
        jax.ShapeDtypeStruct((NW, NSEG), jnp.float32),
    ],
    compiler_params=_sc_params,
    scratch_types=[
        pltpu.VMEM((2, CHUNK, HALF), jnp.float32),
        pltpu.VMEM((2, CHUNK, HALF), jnp.float32),
        pltpu.VMEM((CHUNK,), jnp.int32),
        pltpu.VMEM((CHUNK,), jnp.int32),
        pltpu.VMEM((NSEG,), jnp.float32),
        pltpu.VMEM((CSLICE,), jnp.int32),
        pltpu.VMEM((TROWS, HALF), jnp.float32),
        pltpu.VMEM_SHARED((NSEG, HALF), jnp.float32),
        pltpu.VMEM_SHARED((NSEG, HALF), jnp.float32),
        pltpu.SemaphoreType.DMA,
        pltpu.SemaphoreType.DMA,
    ],
)
def _sc_partial(x_hbm, b_hbm, sums_hbm, cnts_hbm, xl, xr, bbuf0, bbuf1, hist,
                cbuf, zstage, accl, accr, sem0, sem1):
    c = lax.axis_index("c")
    s = lax.axis_index("s")
    w = c * NS + s
    bbufs = (bbuf0, bbuf1)
    sems = (sem0, sem1)

    one = jnp.ones((LANES,), jnp.float32)
    zero = jnp.zeros((LANES,), jnp.float32)
    row = s * TROWS

    k0 = w * NSC // NW
    k1 = (w + 1) * NSC // NW
    cntc = k1 - k0

    def start(k, slot):
        r = RTC + k * CHUNK
        sem = sems[slot]
        cps = (
            pltpu.make_async_copy(
                x_hbm.at[pl.ds(r, CHUNK), pl.ds(0, HALF)], xl.at[slot], sem),
            pltpu.make_async_copy(
                x_hbm.at[pl.ds(r, CHUNK), pl.ds(HALF, HALF)], xr.at[slot], sem),
            pltpu.make_async_copy(b_hbm.at[pl.ds(r, CHUNK)], bbufs[slot], sem),
        )
        for cp in cps:
            cp.start()
        return cps

    def wait(cps):
        for cp in cps:
            cp.wait()

    def accumulate(slot):
        pltpu.sync_copy(xl.at[slot], accl.at[bbufs[slot]], add=True)
        pltpu.sync_copy(xr.at[slot], accr.at[bbufs[slot]], add=True)

        @pl.loop(0, CHUNK // LANES)
        def _(i):
            plsc.addupdate_scatter(
                hist, [bbufs[slot][pl.ds(i * LANES, LANES)]], one)

    cp0 = start(k0, 0)

    @pl.loop(0, NSEG // LANES)
    def _(i):
        hist[pl.ds(i * LANES, LANES)] = zero

    @pl.loop(0, TROWS)
    def _(i):
        @pl.loop(0, HALF // LANES)
        def _(j):
            zstage[i, pl.ds(j * LANES, LANES)] = zero

    cb = w * CSLICE

    @pl.when(w < NW - 1)
    def _():
        pltpu.sync_copy(b_hbm.at[pl.ds(cb, CSLICE)], cbuf)

    @pl.when(w == NW - 1)
    def _():
        pltpu.sync_copy(b_hbm.at[pl.ds(cb, CSLICE_LAST)],
                        cbuf.at[pl.ds(0, CSLICE_LAST)])

    nit = jnp.where(w == NW - 1, CSLICE_LAST // LANES, CSLICE // LANES)

    @pl.loop(0, nit)
    def _(i):
        plsc.addupdate_scatter(hist, [cbuf[pl.ds(i * LANES, LANES)]], one)

    pltpu.sync_copy(zstage, accl.at[pl.ds(row, TROWS)])
    pltpu.sync_copy(zstage, accr.at[pl.ds(row, TROWS)])
    plsc.subcore_barrier()

    npairs = cntc // 2

    @pl.loop(0, npairs)
    def _(p):
        k = k0 + 2 * p
        nxt = start(k + 1, 1)
        wait(cp0)
        accumulate(0)

        @pl.when(k + 2 < k1)
        def _():
            start(k + 2, 0)

        wait(nxt)
        accumulate(1)

    @pl.when(k0 + 2 * npairs < k1)
    def _():
        wait(cp0)
        accumulate(0)

    pltpu.sync_copy(hist, cnts_hbm.at[w])
    plsc.subcore_barrier()
    pltpu.sync_copy(accl.at[pl.ds(row, TROWS)],
                    sums_hbm.at[c, pl.ds(row, TROWS), pl.ds(0, HALF)])
    pltpu.sync_copy(accr.at[pl.ds(row, TROWS)],
                    sums_hbm.at[c, pl.ds(row, TROWS), pl.ds(HALF, HALF)])


def _sums_body(b_ref, x_ref, o_ref):
    i = pl.program_id(0)
    bvec = b_ref[0, 0, :]
    gids = jax.lax.broadcasted_iota(jnp.int32, (NSEG, TCCHUNK), 0)
    onehot = (gids == bvec[None, :]).astype(jnp.bfloat16)
    psum = jax.lax.dot(onehot, x_ref[...].astype(jnp.bfloat16),
                       preferred_element_type=jnp.float32)

    @pl.when(i == 0)
    def _():
        o_ref[...] = psum

    @pl.when(i > 0)
    def _():
        o_ref[...] += psum


def _tc_sums(x, b3):
    return pl.pallas_call(
        _sums_body,
        grid=(NTCCHUNK,),
        in_specs=[
            pl.BlockSpec((1, 1, TCCHUNK), lambda i: (i, 0, 0)),
            pl.BlockSpec((TCCHUNK, FEAT), lambda i: (i, 0)),
        ],
        out_specs=pl.BlockSpec((NSEG, FEAT), lambda i: (0, 0)),
        out_shape=jax.ShapeDtypeStruct((NSEG, FEAT), jnp.float32),
    )(b3, x)


def _combine_body(ts_ref, sp_ref, cp_ref, o_ref):
    ssum = ts_ref[...] + sp_ref[0] + sp_ref[1]
    csum = jnp.sum(cp_ref[...], axis=0).reshape(NSEG, 1)
    o_ref[...] = ssum / jnp.maximum(csum, 1.0)


def kernel(x, batch):
    b = batch.astype(jnp.int32)
    b3 = b.reshape(ROWS // TCCHUNK, 1, TCCHUNK)
    tc_sums = _tc_sums(x, b3)
    sums, cnts = _sc_partial(x, b)
    return pl.pallas_call(
        _combine_body,
        out_shape=jax.ShapeDtypeStruct((NSEG, FEAT), jnp.float32),
    )(tc_sums, sums, cnts)
